# baseline (speedup 1.0000x reference)
import functools

import jax
import jax.numpy as jnp
from jax import lax
from jax.experimental import pallas as pl
from jax.experimental.pallas import tpu as pltpu
from jax.experimental.pallas import tpu_sc as plsc

VOCAB = 100000
EMB = 128
MAX_LEN = 2048
BATCH = 4

_NC = 2
_NS = 16
_NW = _NC * _NS
_N = BATCH * MAX_LEN
_BPW = _N // _NW
_ICHUNK = 128
_NJ = _BPW // _ICHUNK
_SREP = 128

_mesh = plsc.VectorSubcoreMesh(core_axis_name="c", subcore_axis_name="s")


@functools.partial(
    pl.kernel,
    out_type=jax.ShapeDtypeStruct((_N, EMB), jnp.float32),
    mesh=_mesh,
    scratch_types=[
        pltpu.VMEM((_NJ, _ICHUNK), jnp.int32),
        pltpu.VMEM((_NJ, _ICHUNK), jnp.int32),
        pltpu.VMEM((_BPW, EMB), jnp.float32),
        pltpu.VMEM_SHARED((MAX_LEN, EMB), jnp.float32),
        pltpu.SemaphoreType.DMA,
    ],
)
def _emb_kernel(tok_hbm, idx_hbm, sid_hbm, seq_hbm, pos_hbm, out_hbm,
                idx_v, sid_v, acc_v, pos_sh, sem):
    s = lax.axis_index("s")
    wid = s * _NC + lax.axis_index("c")
    base = wid * _BPW
    l0 = lax.rem(base, MAX_LEN)
    b = lax.div(base, MAX_LEN)

    cps = []
    for j in range(_NJ):
        sl = (b, pl.ds(l0 + j * _ICHUNK, _ICHUNK))
        cps.append(pltpu.async_copy(idx_hbm.at[sl], idx_v.at[j], sem))
        cps.append(pltpu.async_copy(sid_hbm.at[sl], sid_v.at[j], sem))
    stage = pl.ds(s * (MAX_LEN // _NS), MAX_LEN // _NS)
    pltpu.sync_copy(pos_hbm.at[stage], pos_sh.at[stage])
    plsc.subcore_barrier()
    cps.append(pltpu.async_copy(pos_sh.at[pl.ds(l0, _BPW)], acc_v, sem))
    for cp in cps:
        cp.wait()

    iota = jnp.arange(16, dtype=jnp.int32)
    for j in range(_NJ):
        for c in range(_ICHUNK // 16):
            sl = (j, pl.ds(c * 16, 16))
            sid_v[sl] = sid_v[sl] + 2 * (iota + c * 16)

    cps = []
    for j in range(_NJ):
        dst = acc_v.at[pl.ds(j * _ICHUNK, _ICHUNK)]
        cps.append(pltpu.async_copy(tok_hbm.at[idx_v.at[j]], dst, sem, add=True))
        cps.append(pltpu.async_copy(seq_hbm.at[sid_v.at[j]], dst, sem, add=True))
    for cp in cps:
        cp.wait()

    pltpu.sync_copy(acc_v, out_hbm.at[pl.ds(base, _BPW)])


def kernel(inputs, sequence_id, token_table, seq_table, pos_table):
    idx = inputs.astype(jnp.int32)
    sid = sequence_id.astype(jnp.int32)
    seq_rep = jnp.tile(seq_table, (_SREP, 1))
    out = _emb_kernel(token_table, idx, sid, seq_rep, pos_table)
    return jnp.reshape(out, (BATCH, MAX_LEN, EMB))


# speedup vs baseline: 1.0127x; 1.0127x over previous
import functools

import jax
import jax.numpy as jnp
from jax import lax
from jax.experimental import pallas as pl
from jax.experimental.pallas import tpu as pltpu
from jax.experimental.pallas import tpu_sc as plsc

VOCAB = 100000
EMB = 128
MAX_LEN = 2048
BATCH = 4

_NC = 2
_NS = 16
_NW = _NC * _NS
_N = BATCH * MAX_LEN
_BPW = _N // _NW
_ICHUNK = 128
_NJ = _BPW // _ICHUNK
_SREP = 128

_mesh = plsc.VectorSubcoreMesh(core_axis_name="c", subcore_axis_name="s")


@functools.partial(
    pl.kernel,
    out_type=jax.ShapeDtypeStruct((_N, EMB), jnp.float32),
    mesh=_mesh,
    scratch_types=[
        pltpu.VMEM((_NJ, _ICHUNK), jnp.int32),
        pltpu.VMEM((_NJ, _ICHUNK), jnp.int32),
        pltpu.VMEM((_BPW, EMB), jnp.float32),
        pltpu.SemaphoreType.DMA,
    ],
)
def _emb_kernel(tok_hbm, idx_hbm, sid_hbm, seq_hbm, pos_hbm, out_hbm,
                idx_v, sid_v, acc_v, sem):
    wid = lax.axis_index("s") * _NC + lax.axis_index("c")
    base = wid * _BPW
    l0 = lax.rem(base, MAX_LEN)

    cps = [
        pltpu.async_copy(idx_hbm.at[pl.ds(wid * _NJ, _NJ)], idx_v, sem),
        pltpu.async_copy(sid_hbm.at[pl.ds(wid * _NJ, _NJ)], sid_v, sem),
        pltpu.async_copy(pos_hbm.at[pl.ds(l0, _BPW)], acc_v, sem),
    ]
    for cp in cps:
        cp.wait()

    iota = jnp.arange(16, dtype=jnp.int32)
    for j in range(_NJ):
        for c in range(_ICHUNK // 16):
            sl = (j, pl.ds(c * 16, 16))
            sid_v[sl] = sid_v[sl] + 2 * (iota + c * 16)

    cps = []
    for j in range(_NJ):
        dst = acc_v.at[pl.ds(j * _ICHUNK, _ICHUNK)]
        cps.append(pltpu.async_copy(tok_hbm.at[idx_v.at[j]], dst, sem, add=True))
        cps.append(pltpu.async_copy(seq_hbm.at[sid_v.at[j]], dst, sem, add=True))
    for cp in cps:
        cp.wait()

    pltpu.sync_copy(acc_v, out_hbm.at[pl.ds(base, _BPW)])


def kernel(inputs, sequence_id, token_table, seq_table, pos_table):
    idx = jnp.reshape(inputs.astype(jnp.int32), (_N // _ICHUNK, _ICHUNK))
    sid = jnp.reshape(sequence_id.astype(jnp.int32), (_N // _ICHUNK, _ICHUNK))
    seq_rep = jnp.tile(seq_table, (_SREP, 1))
    out = _emb_kernel(token_table, idx, sid, seq_rep, pos_table)
    return jnp.reshape(out, (BATCH, MAX_LEN, EMB))


# speedup vs baseline: 1.0222x; 1.0094x over previous
import functools

import jax
import jax.numpy as jnp
from jax import lax
from jax.experimental import pallas as pl
from jax.experimental.pallas import tpu as pltpu
from jax.experimental.pallas import tpu_sc as plsc

VOCAB = 100000
EMB = 128
MAX_LEN = 2048
BATCH = 4

_NC = 2
_NS = 16
_NW = _NC * _NS
_N = BATCH * MAX_LEN
_BPW = _N // _NW
_ICHUNK = 128
_NJ = _BPW // _ICHUNK
_SREP = 128

@functools.cache
def _make_emb_kernel():
    mesh = plsc.VectorSubcoreMesh(core_axis_name="c", subcore_axis_name="s")
    return pl.kernel(
        _emb_body,
        out_type=jax.ShapeDtypeStruct((_N, EMB), jnp.float32),
        mesh=mesh,
        scratch_types=[
            pltpu.VMEM((_NJ, _ICHUNK), jnp.int32),
            pltpu.VMEM((_NJ, _ICHUNK), jnp.int32),
            pltpu.VMEM((_BPW, EMB), jnp.float32),
            pltpu.SemaphoreType.DMA,
        ],
    )


def _emb_body(tok_hbm, idx_hbm, sid_hbm, seq_hbm, pos_hbm, out_hbm,
              idx_v, sid_v, acc_v, sem):
    wid = lax.axis_index("s") * _NC + lax.axis_index("c")
    base = wid * _BPW
    l0 = lax.rem(base, MAX_LEN)

    cps = [
        pltpu.async_copy(idx_hbm.at[pl.ds(wid * _NJ, _NJ)], idx_v, sem),
        pltpu.async_copy(sid_hbm.at[pl.ds(wid * _NJ, _NJ)], sid_v, sem),
        pltpu.async_copy(pos_hbm.at[pl.ds(l0, _BPW)], acc_v, sem),
    ]
    for cp in cps:
        cp.wait()

    iota = jnp.arange(16, dtype=jnp.int32)
    for j in range(_NJ):
        for c in range(_ICHUNK // 16):
            sl = (j, pl.ds(c * 16, 16))
            sid_v[sl] = sid_v[sl] + 2 * (iota + c * 16)

    cps = []
    for j in range(_NJ):
        dst = acc_v.at[pl.ds(j * _ICHUNK, _ICHUNK)]
        cps.append(pltpu.async_copy(tok_hbm.at[idx_v.at[j]], dst, sem, add=True))
        cps.append(pltpu.async_copy(seq_hbm.at[sid_v.at[j]], dst, sem, add=True))
    for cp in cps:
        cp.wait()

    pltpu.sync_copy(acc_v, out_hbm.at[pl.ds(base, _BPW)])


def kernel(inputs, sequence_id, token_table, seq_table, pos_table):
    idx = jnp.reshape(inputs.astype(jnp.int32), (_N // _ICHUNK, _ICHUNK))
    sid = jnp.reshape(sequence_id.astype(jnp.int32), (_N // _ICHUNK, _ICHUNK))
    seq_rep = jnp.tile(seq_table, (_SREP, 1))
    out = _make_emb_kernel()(token_table, idx, sid, seq_rep, pos_table)
    return jnp.reshape(out, (BATCH, MAX_LEN, EMB))
